# baseline (speedup 1.0000x reference)
import jax
import jax.numpy as jnp
from jax.experimental import pallas as pl
from jax.experimental.pallas import tpu as pltpu

B, S, F = 64, 25, 256
NE, NH = 10, 4
LK = 32
SP = 32
NB = 32


def _fold_cols(w, avec_ref):
    cols = []
    for h in range(NH):
        wh = w[:, h * F:(h + 1) * F]
        cols.append(jax.lax.dot_general(
            wh, avec_ref[h:h + 1, :], (((1,), (1,)), ((), ())),
            preferred_element_type=jnp.float32))
    return cols


def _one_modality(x3, mjf_s, w, cst, bias, hsel, ksel, base_f, cl2_f, o_ref):
    x3p = jnp.concatenate(
        [x3, jnp.zeros((NB, SP - S, F), jnp.float32)], axis=1)
    x2 = x3p.reshape(NB * SP, F)
    proj2 = jnp.dot(x2.astype(jnp.bfloat16), w.astype(jnp.bfloat16),
                    preferred_element_type=jnp.float32)
    proj3 = proj2.reshape(NB, SP, NH * F).astype(jnp.bfloat16)
    sst2 = jnp.dot(x2, cst, preferred_element_type=jnp.float32)
    sst3 = sst2.reshape(NB, SP, 2 * NH)

    mjf = jnp.concatenate(
        [mjf_s, jnp.zeros((NB, SP - S, NE), jnp.float32)], axis=1)
    mkf = jax.lax.dot_general(
        mjf, ksel, (((1,), (0,)), ((), ())),
        preferred_element_type=jnp.float32)

    ss_pack = jax.lax.dot_general(
        sst3[:, :, :NH], hsel, (((2,), (0,)), ((), ())),
        preferred_element_type=jnp.float32)
    st_rows = jax.lax.dot_general(
        sst3[:, :, NH:], ksel, (((1,), (0,)), ((), ())),
        preferred_element_type=jnp.float32)
    h_1 = jax.lax.broadcasted_iota(jnp.int32, (NB, 1, NH * LK), 2) // LK
    st_pack = jnp.zeros((NB, 1, NH * LK), jnp.float32)
    for h in range(NH):
        st_pack = jnp.where(h_1 == h, st_rows[:, h:h + 1, :], st_pack)

    sc = ss_pack + st_pack
    sc = jnp.where(sc >= 0, sc, 0.2 * sc)
    cmax = jnp.max(sc, axis=1, keepdims=True)
    e = jnp.exp(sc - cmax)

    g = e * cl2_f
    den_base = jnp.sum(e * base_f, axis=1, keepdims=True)
    d_cl = jax.lax.dot_general(
        mjf, g, (((1,), (1,)), ((0,), (0,))),
        preferred_element_type=jnp.float32)
    r = 1.0 / (den_base + mkf * d_cl + 1e-16)
    rk = mkf * r
    rsum = jnp.sum(r, axis=1, keepdims=True)
    s2 = jax.lax.dot_general(
        mjf, rk, (((2,), (1,)), ((0,), (0,))),
        preferred_element_type=jnp.float32)
    atot = e * (base_f * rsum + cl2_f * s2)

    acc = x3p + bias
    atot16 = atot.astype(jnp.bfloat16)
    for h in range(NH):
        acc = acc + (1.0 / (NE * NH)) * jax.lax.dot_general(
            atot16[:, :, h * LK:(h + 1) * LK], proj3[:, :, h * F:(h + 1) * F],
            (((1,), (1,)), ((0,), (0,))),
            preferred_element_type=jnp.float32)
    o_ref[...] = acc[:, :S, :]


def _gat_body(vx_ref, ax_ref, vp_ref, ap_ref, thr_ref, vw_ref, vsrc_ref,
              vtrg_ref, vb_ref, aw_ref, asrc_ref, atrg_ref, ab_ref,
              vo_ref, ao_ref, cst_ref):
    j2 = jax.lax.broadcasted_iota(jnp.int32, (SP, NH * LK), 0)
    l2 = jax.lax.broadcasted_iota(jnp.int32, (SP, NH * LK), 1)
    k2 = jnp.bitwise_and(l2, LK - 1)
    valid2 = (k2 < S) & (j2 < S)
    base2 = valid2 & ((jnp.abs(j2 - k2) == 1) | (j2 == k2))
    base_f = base2.astype(jnp.float32)
    cl2_f = (valid2 & jnp.logical_not(base2)).astype(jnp.float32)
    hsel = (jax.lax.broadcasted_iota(jnp.int32, (NH, NH * LK), 1) // LK ==
            jax.lax.broadcasted_iota(jnp.int32, (NH, NH * LK), 0)).astype(jnp.float32)
    ksel = (k2 == j2).astype(jnp.float32)

    thr = thr_ref[0, 0]
    mo = pl.program_id(1)
    first = pl.program_id(0) == 0

    @pl.when(mo == 0)
    def _video():
        vw = vw_ref[...]

        @pl.when(first)
        def _fold_v():
            cst_ref[:, :2 * NH] = jnp.concatenate(
                _fold_cols(vw, vsrc_ref) + _fold_cols(vw, vtrg_ref), axis=1)

        _one_modality(vx_ref[...], (vp_ref[...] >= thr).astype(jnp.float32),
                      vw, cst_ref[:, :2 * NH], vb_ref[...], hsel, ksel,
                      base_f, cl2_f, vo_ref)

    @pl.when(mo == 1)
    def _audio():
        aw = aw_ref[...]

        @pl.when(first)
        def _fold_a():
            cst_ref[:, 2 * NH:] = jnp.concatenate(
                _fold_cols(aw, asrc_ref) + _fold_cols(aw, atrg_ref), axis=1)

        _one_modality(ax_ref[...], (ap_ref[...] >= thr).astype(jnp.float32),
                      aw, cst_ref[:, 2 * NH:], ab_ref[...], hsel, ksel,
                      base_f, cl2_f, ao_ref)


def kernel(video_features, audio_features, video_snippet_preds,
           audio_snippet_preds, confidence_threshold, aW0, a_src0, a_trg0,
           a_b0, vW0, v_src0, v_trg0, v_b0):
    thr = jnp.asarray(confidence_threshold, jnp.float32).reshape(1, 1)
    nb_blocks = B // NB
    blk = lambda b, mo: (b, 0, 0)
    fix2 = lambda b, mo: (0, 0)
    vo, ao = pl.pallas_call(
        _gat_body,
        grid=(nb_blocks, 2),
        in_specs=[
            pl.BlockSpec((NB, S, F), blk),
            pl.BlockSpec((NB, S, F), blk),
            pl.BlockSpec((NB, S, NE), blk),
            pl.BlockSpec((NB, S, NE), blk),
            pl.BlockSpec((1, 1), fix2),
            pl.BlockSpec((F, NH * F), fix2),
            pl.BlockSpec((NH, F), fix2),
            pl.BlockSpec((NH, F), fix2),
            pl.BlockSpec((1, F), fix2),
            pl.BlockSpec((F, NH * F), fix2),
            pl.BlockSpec((NH, F), fix2),
            pl.BlockSpec((NH, F), fix2),
            pl.BlockSpec((1, F), fix2),
        ],
        out_specs=[pl.BlockSpec((NB, S, F), blk), pl.BlockSpec((NB, S, F), blk)],
        out_shape=[jax.ShapeDtypeStruct((B, S, F), jnp.float32),
                   jax.ShapeDtypeStruct((B, S, F), jnp.float32)],
        scratch_shapes=[pltpu.VMEM((F, 4 * NH), jnp.float32)],
    )(video_features, audio_features, video_snippet_preds, audio_snippet_preds,
      thr, vW0, v_src0, v_trg0, v_b0.reshape(1, F), aW0, a_src0, a_trg0,
      a_b0.reshape(1, F))
    return (vo, ao)


# speedup vs baseline: 1.0585x; 1.0585x over previous
import jax
import jax.numpy as jnp
from jax.experimental import pallas as pl
from jax.experimental.pallas import tpu as pltpu

B, S, F = 64, 25, 256
NE, NH = 10, 4
LK = 32
SP = 32
NB = 32


def _fold_cols(w, avec_ref):
    cols = []
    for h in range(NH):
        wh = w[:, h * F:(h + 1) * F]
        cols.append(jax.lax.dot_general(
            wh, avec_ref[h:h + 1, :], (((1,), (1,)), ((), ())),
            preferred_element_type=jnp.float32))
    return cols


def _one_modality(x3, mjf_s, w16, cst, bias, hsel, ksel, base_f, cl2_f, o_ref):
    x3p = jnp.concatenate(
        [x3, jnp.zeros((NB, SP - S, F), jnp.float32)], axis=1)
    x2 = x3p.reshape(NB * SP, F)
    proj2 = jnp.dot(x2.astype(jnp.bfloat16), w16,
                    preferred_element_type=jnp.float32)
    proj3 = proj2.reshape(NB, SP, NH * F).astype(jnp.bfloat16)
    sst2 = jnp.dot(x2, cst, preferred_element_type=jnp.float32)
    sst3 = sst2.reshape(NB, SP, 2 * NH)
    ss_pack2 = jnp.dot(sst2[:, :NH], hsel,
                       preferred_element_type=jnp.float32)

    mjf = jnp.concatenate(
        [mjf_s, jnp.zeros((NB, SP - S, NE), jnp.float32)], axis=1)
    mkf = jax.lax.dot_general(
        mjf, ksel, (((1,), (0,)), ((), ())),
        preferred_element_type=jnp.float32)

    ss_pack = ss_pack2.reshape(NB, SP, NH * LK)
    st_rows = jax.lax.dot_general(
        sst3[:, :, NH:], ksel, (((1,), (0,)), ((), ())),
        preferred_element_type=jnp.float32)
    h_1 = jax.lax.broadcasted_iota(jnp.int32, (NB, 1, NH * LK), 2) // LK
    st_pack = jnp.zeros((NB, 1, NH * LK), jnp.float32)
    for h in range(NH):
        st_pack = jnp.where(h_1 == h, st_rows[:, h:h + 1, :], st_pack)

    sc = ss_pack + st_pack
    sc = jnp.where(sc >= 0, sc, 0.2 * sc)
    cmax = jnp.max(sc, axis=1, keepdims=True)
    e = jnp.exp(sc - cmax)

    g = e * cl2_f
    den_base = jnp.sum(e * base_f, axis=1, keepdims=True)
    d_cl = jax.lax.dot_general(
        mjf, g, (((1,), (1,)), ((0,), (0,))),
        preferred_element_type=jnp.float32)
    r = 1.0 / (den_base + mkf * d_cl + 1e-16)
    rk = mkf * r
    rsum = jnp.sum(r, axis=1, keepdims=True)
    s2 = jax.lax.dot_general(
        mjf, rk, (((2,), (1,)), ((0,), (0,))),
        preferred_element_type=jnp.float32)
    atot = e * (base_f * rsum + cl2_f * s2)

    acc = x3p + bias
    atot16 = atot.astype(jnp.bfloat16)
    for h in range(NH):
        acc = acc + (1.0 / (NE * NH)) * jax.lax.dot_general(
            atot16[:, :, h * LK:(h + 1) * LK], proj3[:, :, h * F:(h + 1) * F],
            (((1,), (1,)), ((0,), (0,))),
            preferred_element_type=jnp.float32)
    o_ref[...] = acc[:, :S, :]


def _gat_body(vx_ref, ax_ref, vp_ref, ap_ref, thr_ref, vw_ref, vsrc_ref,
              vtrg_ref, vb_ref, aw_ref, asrc_ref, atrg_ref, ab_ref,
              vw16_ref, aw16_ref, vo_ref, ao_ref, cst_ref):
    vw = vw_ref[...]
    aw = aw_ref[...]

    @pl.when(pl.program_id(0) == 0)
    def _fold():
        cols = (_fold_cols(vw, vsrc_ref) + _fold_cols(vw, vtrg_ref) +
                _fold_cols(aw, asrc_ref) + _fold_cols(aw, atrg_ref))
        cst_ref[...] = jnp.concatenate(cols, axis=1)

    j2 = jax.lax.broadcasted_iota(jnp.int32, (SP, NH * LK), 0)
    l2 = jax.lax.broadcasted_iota(jnp.int32, (SP, NH * LK), 1)
    k2 = jnp.bitwise_and(l2, LK - 1)
    valid2 = (k2 < S) & (j2 < S)
    base2 = valid2 & ((jnp.abs(j2 - k2) == 1) | (j2 == k2))
    base_f = base2.astype(jnp.float32)
    cl2_f = (valid2 & jnp.logical_not(base2)).astype(jnp.float32)
    hsel = (jax.lax.broadcasted_iota(jnp.int32, (NH, NH * LK), 1) // LK ==
            jax.lax.broadcasted_iota(jnp.int32, (NH, NH * LK), 0)).astype(jnp.float32)
    ksel = (k2 == j2).astype(jnp.float32)

    thr = thr_ref[0, 0]
    _one_modality(vx_ref[...], (vp_ref[...] >= thr).astype(jnp.float32),
                  vw16_ref[...], cst_ref[:, :2 * NH], vb_ref[...], hsel, ksel,
                  base_f, cl2_f, vo_ref)
    _one_modality(ax_ref[...], (ap_ref[...] >= thr).astype(jnp.float32),
                  aw16_ref[...], cst_ref[:, 2 * NH:], ab_ref[...], hsel, ksel,
                  base_f, cl2_f, ao_ref)


def kernel(video_features, audio_features, video_snippet_preds,
           audio_snippet_preds, confidence_threshold, aW0, a_src0, a_trg0,
           a_b0, vW0, v_src0, v_trg0, v_b0):
    thr = jnp.asarray(confidence_threshold, jnp.float32).reshape(1, 1)
    nb_blocks = B // NB
    blk = lambda b: (b, 0, 0)
    fix3 = lambda b: (0, 0, 0)
    fix2 = lambda b: (0, 0)
    vo, ao = pl.pallas_call(
        _gat_body,
        grid=(nb_blocks,),
        in_specs=[
            pl.BlockSpec((NB, S, F), blk),
            pl.BlockSpec((NB, S, F), blk),
            pl.BlockSpec((NB, S, NE), blk),
            pl.BlockSpec((NB, S, NE), blk),
            pl.BlockSpec((1, 1), fix2),
            pl.BlockSpec((F, NH * F), fix2),
            pl.BlockSpec((NH, F), fix2),
            pl.BlockSpec((NH, F), fix2),
            pl.BlockSpec((1, F), fix2),
            pl.BlockSpec((F, NH * F), fix2),
            pl.BlockSpec((NH, F), fix2),
            pl.BlockSpec((NH, F), fix2),
            pl.BlockSpec((1, F), fix2),
            pl.BlockSpec((F, NH * F), fix2),
            pl.BlockSpec((F, NH * F), fix2),
        ],
        out_specs=[pl.BlockSpec((NB, S, F), blk), pl.BlockSpec((NB, S, F), blk)],
        out_shape=[jax.ShapeDtypeStruct((B, S, F), jnp.float32),
                   jax.ShapeDtypeStruct((B, S, F), jnp.float32)],
        scratch_shapes=[pltpu.VMEM((F, 4 * NH), jnp.float32)],
    )(video_features, audio_features, video_snippet_preds, audio_snippet_preds,
      thr, vW0, v_src0, v_trg0, v_b0.reshape(1, F), aW0, a_src0, a_trg0,
      a_b0.reshape(1, F), vW0.astype(jnp.bfloat16), aW0.astype(jnp.bfloat16))
    return (vo, ao)


# speedup vs baseline: 1.0747x; 1.0153x over previous
import jax
import jax.numpy as jnp
from jax.experimental import pallas as pl
from jax.experimental.pallas import tpu as pltpu

B, S, F = 64, 25, 256
NE, NH = 10, 4
LK = 32
SP = 32
NB = 32


def _fold_cols(w, avec_ref):
    cols = []
    for h in range(NH):
        wh = w[:, h * F:(h + 1) * F]
        cols.append(jax.lax.dot_general(
            wh, avec_ref[h:h + 1, :], (((1,), (1,)), ((), ())),
            preferred_element_type=jnp.float32))
    return cols


def _one_modality(x3, mjf_s, w, cst, bias, hsel, ksel, base_f, cl2_f, o_ref):
    x3p = jnp.concatenate(
        [x3, jnp.zeros((NB, SP - S, F), jnp.float32)], axis=1)
    x2 = x3p.reshape(NB * SP, F)
    proj2 = jnp.dot(x2.astype(jnp.bfloat16), w.astype(jnp.bfloat16),
                    preferred_element_type=jnp.float32)
    proj3 = proj2.reshape(NB, SP, NH * F).astype(jnp.bfloat16)
    sst2 = jnp.dot(x2, cst, preferred_element_type=jnp.float32)
    sst3 = sst2.reshape(NB, SP, 2 * NH)

    mjf = jnp.concatenate(
        [mjf_s, jnp.zeros((NB, SP - S, NE), jnp.float32)], axis=1)
    mkf = jax.lax.dot_general(
        mjf, ksel, (((1,), (0,)), ((), ())),
        preferred_element_type=jnp.float32)

    ss_pack = jax.lax.dot_general(
        sst3[:, :, :NH], hsel, (((2,), (0,)), ((), ())),
        preferred_element_type=jnp.float32)
    st_rows = jax.lax.dot_general(
        sst3[:, :, NH:], ksel, (((1,), (0,)), ((), ())),
        preferred_element_type=jnp.float32)
    h_1 = jax.lax.broadcasted_iota(jnp.int32, (NB, 1, NH * LK), 2) // LK
    st_pack = jnp.zeros((NB, 1, NH * LK), jnp.float32)
    for h in range(NH):
        st_pack = jnp.where(h_1 == h, st_rows[:, h:h + 1, :], st_pack)

    sc = ss_pack + st_pack
    sc = jnp.where(sc >= 0, sc, 0.2 * sc)
    cmax = jnp.max(sc, axis=1, keepdims=True)
    e = jnp.exp(sc - cmax)

    g = e * cl2_f
    den_base = jnp.sum(e * base_f, axis=1, keepdims=True)
    d_cl = jax.lax.dot_general(
        mjf, g, (((1,), (1,)), ((0,), (0,))),
        preferred_element_type=jnp.float32)
    r = 1.0 / (den_base + mkf * d_cl + 1e-16)
    rk = mkf * r
    rsum = jnp.sum(r, axis=1, keepdims=True)
    s2 = jax.lax.dot_general(
        mjf, rk, (((2,), (1,)), ((0,), (0,))),
        preferred_element_type=jnp.float32)
    atot = e * (base_f * rsum + cl2_f * s2)

    acc = x3p + bias
    atot16 = atot.astype(jnp.bfloat16)
    for h in range(NH):
        acc = acc + (1.0 / (NE * NH)) * jax.lax.dot_general(
            atot16[:, :, h * LK:(h + 1) * LK], proj3[:, :, h * F:(h + 1) * F],
            (((1,), (1,)), ((0,), (0,))),
            preferred_element_type=jnp.float32)
    o_ref[...] = acc[:, :S, :]


def _gat_body(vx_ref, ax_ref, vp_ref, ap_ref, thr_ref, vw_ref, vsrc_ref,
              vtrg_ref, vb_ref, aw_ref, asrc_ref, atrg_ref, ab_ref,
              vo_ref, ao_ref, cst_ref):
    vw = vw_ref[...]
    aw = aw_ref[...]

    @pl.when(pl.program_id(0) == 0)
    def _fold():
        cols = (_fold_cols(vw, vsrc_ref) + _fold_cols(vw, vtrg_ref) +
                _fold_cols(aw, asrc_ref) + _fold_cols(aw, atrg_ref))
        cst_ref[...] = jnp.concatenate(cols, axis=1)

    j2 = jax.lax.broadcasted_iota(jnp.int32, (SP, NH * LK), 0)
    l2 = jax.lax.broadcasted_iota(jnp.int32, (SP, NH * LK), 1)
    k2 = jnp.bitwise_and(l2, LK - 1)
    valid2 = (k2 < S) & (j2 < S)
    base2 = valid2 & ((jnp.abs(j2 - k2) == 1) | (j2 == k2))
    base_f = base2.astype(jnp.float32)
    cl2_f = (valid2 & jnp.logical_not(base2)).astype(jnp.float32)
    hsel = (jax.lax.broadcasted_iota(jnp.int32, (NH, NH * LK), 1) // LK ==
            jax.lax.broadcasted_iota(jnp.int32, (NH, NH * LK), 0)).astype(jnp.float32)
    ksel = (k2 == j2).astype(jnp.float32)

    thr = thr_ref[0, 0]
    _one_modality(vx_ref[...], (vp_ref[...] >= thr).astype(jnp.float32),
                  vw, cst_ref[:, :2 * NH], vb_ref[...], hsel, ksel,
                  base_f, cl2_f, vo_ref)
    _one_modality(ax_ref[...], (ap_ref[...] >= thr).astype(jnp.float32),
                  aw, cst_ref[:, 2 * NH:], ab_ref[...], hsel, ksel,
                  base_f, cl2_f, ao_ref)


def kernel(video_features, audio_features, video_snippet_preds,
           audio_snippet_preds, confidence_threshold, aW0, a_src0, a_trg0,
           a_b0, vW0, v_src0, v_trg0, v_b0):
    thr = jnp.asarray(confidence_threshold, jnp.float32).reshape(1, 1)
    nb_blocks = B // NB
    blk = lambda b: (b, 0, 0)
    fix3 = lambda b: (0, 0, 0)
    fix2 = lambda b: (0, 0)
    vo, ao = pl.pallas_call(
        _gat_body,
        grid=(nb_blocks,),
        in_specs=[
            pl.BlockSpec((NB, S, F), blk),
            pl.BlockSpec((NB, S, F), blk),
            pl.BlockSpec((NB, S, NE), blk),
            pl.BlockSpec((NB, S, NE), blk),
            pl.BlockSpec((1, 1), fix2),
            pl.BlockSpec((F, NH * F), fix2),
            pl.BlockSpec((NH, F), fix2),
            pl.BlockSpec((NH, F), fix2),
            pl.BlockSpec((1, F), fix2),
            pl.BlockSpec((F, NH * F), fix2),
            pl.BlockSpec((NH, F), fix2),
            pl.BlockSpec((NH, F), fix2),
            pl.BlockSpec((1, F), fix2),
        ],
        out_specs=[pl.BlockSpec((NB, S, F), blk), pl.BlockSpec((NB, S, F), blk)],
        out_shape=[jax.ShapeDtypeStruct((B, S, F), jnp.float32),
                   jax.ShapeDtypeStruct((B, S, F), jnp.float32)],
        scratch_shapes=[pltpu.VMEM((F, 4 * NH), jnp.float32)],
    )(video_features, audio_features, video_snippet_preds, audio_snippet_preds,
      thr, vW0, v_src0, v_trg0, v_b0.reshape(1, F), aW0, a_src0, a_trg0,
      a_b0.reshape(1, F))
    return (vo, ao)
